# Initial kernel scaffold; baseline (speedup 1.0000x reference)
import jax
import jax.numpy as jnp
from jax.experimental import pallas as pl


def kernel(x, edge_index, W0, a_src0, a_dst0, b0, W1, a_src1, a_dst1, b1):
    raise NotImplementedError("write your pallas kernel here")


# speedup vs baseline: 47.0903x; 47.0903x over previous
import functools

import jax
import jax.numpy as jnp
from jax import lax
from jax.experimental import pallas as pl
from jax.experimental.pallas import tpu as pltpu
from jax.experimental.pallas import tpu_sc as plsc

N = 10000
DIN = 128
HEADS = 4
HID = 32
DOUT = 128

N1 = 10112
TRASH = N
BLK = 128
EB = 2592
ETOT = EB * BLK
NW = 32
BPW = EB // NW
RPT = N1 // 16

TW = 8

_f32 = jnp.float32
_i32 = jnp.int32


def _bcast_lane(v, i):
    idx = jnp.full((16, 1), i, dtype=_i32)
    dn = lax.GatherDimensionNumbers(
        offset_dims=(), collapsed_slice_dims=(0,), start_index_map=(0,))
    return lax.gather(v, idx, dn, (1,),
                      mode=lax.GatherScatterMode.PROMISE_IN_BOUNDS)


def _make_sc_gat(H):
    mesh = plsc.VectorSubcoreMesh(core_axis_name="c", subcore_axis_name="s")

    @functools.partial(
        pl.kernel,
        out_type=(jax.ShapeDtypeStruct((2, N1, 128), _f32),
                  jax.ShapeDtypeStruct((2, N1, TW), _f32)),
        mesh=mesh,
        compiler_params=pltpu.CompilerParams(
            needs_layout_passes=False, use_tc_tiling_on_sc=False),
        scratch_types=[
            pltpu.VMEM((BPW, BLK), _i32),
            pltpu.VMEM((BPW, BLK), _i32),
            pltpu.VMEM((BLK, 128), _f32),
            pltpu.VMEM((BLK, TW), _f32),
            pltpu.VMEM((BLK, TW), _f32),
            pltpu.VMEM((BLK, TW), _f32),
            pltpu.VMEM_SHARED((N1, 128), _f32),
            pltpu.VMEM_SHARED((N1, TW), _f32),
            pltpu.SemaphoreType.DMA,
        ],
    )
    def sc_gat(src_hbm, dst_hbm, asrc_hbm, adst_hbm, h_hbm,
               op_hbm, dp_hbm,
               sidx_v, didx_v, rows_v, e_v, as_v, ad_v, acc_sh, den_sh,
               sem):
        core = lax.axis_index("c")
        sub = lax.axis_index("s")
        w = sub * 2 + core
        r0 = sub * RPT

        lane = lax.iota(_i32, 16)
        zero16 = jnp.zeros((16,), _f32)

        def zrow(r, c):
            for q in range(8):
                rows_v[r, pl.ds(q * 16, 16)] = zero16
            return c
        lax.fori_loop(0, BLK, zrow, 0)

        def zev(g, c):
            for h in range(TW):
                plsc.store_scatter(
                    e_v, [g * 16 + lane, jnp.full((16,), h, _i32)], zero16)
            return c
        lax.fori_loop(0, 8, zev, 0)

        for t in range(RPT // BLK):
            pltpu.sync_copy(rows_v, acc_sh.at[pl.ds(r0 + t * BLK, BLK)])
            pltpu.sync_copy(e_v, den_sh.at[pl.ds(r0 + t * BLK, BLK)])
        rem = RPT % BLK
        if rem:
            t0 = r0 + (RPT // BLK) * BLK
            pltpu.sync_copy(rows_v.at[pl.ds(0, rem)],
                            acc_sh.at[pl.ds(t0, rem)])
            pltpu.sync_copy(e_v.at[pl.ds(0, rem)],
                            den_sh.at[pl.ds(t0, rem)])

        pltpu.sync_copy(src_hbm.at[w], sidx_v)
        pltpu.sync_copy(dst_hbm.at[w], didx_v)
        plsc.subcore_barrier()

        def blk_body(j, carry):
            cp_h = pltpu.async_copy(h_hbm.at[sidx_v.at[j]], rows_v, sem)
            cp_as = pltpu.async_copy(asrc_hbm.at[sidx_v.at[j]], as_v, sem)
            cp_ad = pltpu.async_copy(adst_hbm.at[didx_v.at[j]], ad_v, sem)
            cp_h.wait()
            cp_as.wait()
            cp_ad.wait()

            def grp_body(g, c2):
                base = g * 16
                hsplat = [jnp.full((16,), h, _i32) for h in range(H)]
                evs = []
                for h in range(H):
                    av = plsc.load_gather(as_v, [base + lane, hsplat[h]])
                    bv = plsc.load_gather(ad_v, [base + lane, hsplat[h]])
                    al = av + bv
                    al = jnp.where(al > 0, al, 0.2 * al)
                    ev = jnp.exp(al)
                    evs.append(ev)
                    plsc.store_scatter(e_v, [base + lane, hsplat[h]], ev)
                for i in range(16):
                    gi = base + i
                    for h in range(H):
                        sc = _bcast_lane(evs[h], i)
                        c0 = h * (128 // H)
                        for q in range(128 // H // 16):
                            cq = c0 + q * 16
                            rows_v[gi, pl.ds(cq, 16)] = (
                                rows_v[gi, pl.ds(cq, 16)] * sc)
                return c2

            lax.fori_loop(0, 8, grp_body, 0)
            pltpu.sync_copy(rows_v, acc_sh.at[didx_v.at[j]], add=True)
            pltpu.sync_copy(e_v, den_sh.at[didx_v.at[j]], add=True)
            return carry

        lax.fori_loop(0, BPW, blk_body, 0)
        plsc.subcore_barrier()

        for t in range(RPT // BLK):
            rr = r0 + t * BLK
            pltpu.sync_copy(acc_sh.at[pl.ds(rr, BLK)], rows_v)
            pltpu.sync_copy(rows_v, op_hbm.at[core, pl.ds(rr, BLK)])
            pltpu.sync_copy(den_sh.at[pl.ds(rr, BLK)], e_v)
            pltpu.sync_copy(e_v, dp_hbm.at[core, pl.ds(rr, BLK)])
        if rem:
            t0 = r0 + (RPT // BLK) * BLK
            pltpu.sync_copy(acc_sh.at[pl.ds(t0, rem)],
                            rows_v.at[pl.ds(0, rem)])
            pltpu.sync_copy(rows_v.at[pl.ds(0, rem)],
                            op_hbm.at[core, pl.ds(t0, rem)])
            pltpu.sync_copy(den_sh.at[pl.ds(t0, rem)],
                            e_v.at[pl.ds(0, rem)])
            pltpu.sync_copy(e_v.at[pl.ds(0, rem)],
                            dp_hbm.at[core, pl.ds(t0, rem)])

    return sc_gat


_sc_gat4 = _make_sc_gat(4)
_sc_gat1 = _make_sc_gat(1)


_BR = 1264


def _lin0_body(x_ref, w_ref, ms_ref, md_ref, h_ref, as_ref, ad_ref):
    h = jnp.dot(x_ref[...], w_ref[...], preferred_element_type=_f32)
    h_ref[...] = h
    as_ref[...] = jnp.dot(h, ms_ref[...], preferred_element_type=_f32)
    ad_ref[...] = jnp.dot(h, md_ref[...], preferred_element_type=_f32)


def _mid_body(op_ref, dp_ref, b_ref, w_ref, ms_ref, md_ref, ex_ref,
              h_ref, as_ref, ad_ref):
    i = pl.program_id(0)
    num = op_ref[0] + op_ref[1]
    den = jnp.dot(dp_ref[0] + dp_ref[1], ex_ref[...],
                  preferred_element_type=_f32)
    s = num / (den + 1e-16) + b_ref[...]
    s = jnp.maximum(s, 0.0)
    rows = i * _BR + lax.broadcasted_iota(_i32, (_BR, 128), 0)
    s = jnp.where(rows < N, s, 0.0)
    h = jnp.dot(s, w_ref[...], preferred_element_type=_f32)
    h_ref[...] = h
    as_ref[...] = jnp.dot(h, ms_ref[...], preferred_element_type=_f32)
    ad_ref[...] = jnp.dot(h, md_ref[...], preferred_element_type=_f32)


def _fin_body(op_ref, dp_ref, b_ref, ex_ref, y_ref):
    num = op_ref[0] + op_ref[1]
    den = jnp.dot(dp_ref[0] + dp_ref[1], ex_ref[...],
                  preferred_element_type=_f32)
    y = num / (den + 1e-16) + b_ref[...]
    y_ref[...] = jnp.maximum(y, 0.0)


def _lin0(xpad, W0, Ms, Md):
    return pl.pallas_call(
        _lin0_body,
        grid=(N1 // _BR,),
        in_specs=[
            pl.BlockSpec((_BR, 128), lambda i: (i, 0)),
            pl.BlockSpec((128, 128), lambda i: (0, 0)),
            pl.BlockSpec((128, 8), lambda i: (0, 0)),
            pl.BlockSpec((128, 8), lambda i: (0, 0)),
        ],
        out_specs=[
            pl.BlockSpec((_BR, 128), lambda i: (i, 0)),
            pl.BlockSpec((_BR, 8), lambda i: (i, 0)),
            pl.BlockSpec((_BR, 8), lambda i: (i, 0)),
        ],
        out_shape=[
            jax.ShapeDtypeStruct((N1, 128), _f32),
            jax.ShapeDtypeStruct((N1, 8), _f32),
            jax.ShapeDtypeStruct((N1, 8), _f32),
        ],
    )(xpad, W0, Ms, Md)


def _mid(op, dp, b0, W1, Ms, Md, ex):
    return pl.pallas_call(
        _mid_body,
        grid=(N1 // _BR,),
        in_specs=[
            pl.BlockSpec((2, _BR, 128), lambda i: (0, i, 0)),
            pl.BlockSpec((2, _BR, 8), lambda i: (0, i, 0)),
            pl.BlockSpec((1, 128), lambda i: (0, 0)),
            pl.BlockSpec((128, 128), lambda i: (0, 0)),
            pl.BlockSpec((128, 8), lambda i: (0, 0)),
            pl.BlockSpec((128, 8), lambda i: (0, 0)),
            pl.BlockSpec((8, 128), lambda i: (0, 0)),
        ],
        out_specs=[
            pl.BlockSpec((_BR, 128), lambda i: (i, 0)),
            pl.BlockSpec((_BR, 8), lambda i: (i, 0)),
            pl.BlockSpec((_BR, 8), lambda i: (i, 0)),
        ],
        out_shape=[
            jax.ShapeDtypeStruct((N1, 128), _f32),
            jax.ShapeDtypeStruct((N1, 8), _f32),
            jax.ShapeDtypeStruct((N1, 8), _f32),
        ],
    )(op, dp, b0, W1, Ms, Md, ex)


def _fin(op, dp, b1, ex):
    return pl.pallas_call(
        _fin_body,
        grid=(10,),
        in_specs=[
            pl.BlockSpec((2, 1000, 128), lambda i: (0, i, 0)),
            pl.BlockSpec((2, 1000, 8), lambda i: (0, i, 0)),
            pl.BlockSpec((1, 128), lambda i: (0, 0)),
            pl.BlockSpec((8, 128), lambda i: (0, 0)),
        ],
        out_specs=pl.BlockSpec((1000, 128), lambda i: (i, 0)),
        out_shape=jax.ShapeDtypeStruct((N, 128), _f32),
    )(op, dp, b1, ex)


def kernel(x, edge_index, W0, a_src0, a_dst0, b0, W1, a_src1, a_dst1, b1):
    loops = jnp.arange(N, dtype=edge_index.dtype)
    pad = ETOT - (edge_index.shape[1] + N)
    src = jnp.concatenate(
        [edge_index[0], loops, jnp.zeros((pad,), edge_index.dtype)])
    dst = jnp.concatenate(
        [edge_index[1], loops, jnp.full((pad,), TRASH, edge_index.dtype)])
    src2d = src.reshape(NW, BPW, BLK).astype(_i32)
    dst2d = dst.reshape(NW, BPW, BLK).astype(_i32)

    xpad = jnp.concatenate([x, jnp.zeros((N1 - N, DIN), _f32)])

    hcol = jnp.repeat(jnp.arange(HEADS), HID)
    Ms0 = jnp.zeros((128, 8), _f32).at[jnp.arange(128), hcol].set(
        a_src0.reshape(-1))
    Md0 = jnp.zeros((128, 8), _f32).at[jnp.arange(128), hcol].set(
        a_dst0.reshape(-1))
    Ms1 = jnp.zeros((128, 8), _f32).at[:, 0].set(a_src1[0])
    Md1 = jnp.zeros((128, 8), _f32).at[:, 0].set(a_dst1[0])
    ex4 = jnp.zeros((8, 128), _f32).at[hcol, jnp.arange(128)].set(1.0)
    ex1 = jnp.zeros((8, 128), _f32).at[0].set(1.0)

    h0, as0, ad0 = _lin0(xpad, W0, Ms0, Md0)
    op0, dp0 = _sc_gat4(src2d, dst2d, as0, ad0, h0)

    h1, as1, ad1 = _mid(op0, dp0, b0.reshape(1, 128), W1, Ms1, Md1, ex4)
    op1, dp1 = _sc_gat1(src2d, dst2d, as1, ad1, h1)

    return _fin(op1, dp1, b1.reshape(1, 128), ex1)


# speedup vs baseline: 63.3159x; 1.3446x over previous
import functools

import jax
import jax.numpy as jnp
from jax import lax
from jax.experimental import pallas as pl
from jax.experimental.pallas import tpu as pltpu
from jax.experimental.pallas import tpu_sc as plsc

N = 10000
DIN = 128
HEADS = 4
HID = 32
DOUT = 128

N1 = 10112
TRASH = N
BLK = 64
NCH = 162
ETOT = 331776
NW = 32
BPW = NCH
RPT = N1 // 16

TW = 8

_f32 = jnp.float32
_i32 = jnp.int32


def _bcast_lane(v, i):
    idx = jnp.full((16, 1), i, dtype=_i32)
    dn = lax.GatherDimensionNumbers(
        offset_dims=(), collapsed_slice_dims=(0,), start_index_map=(0,))
    return lax.gather(v, idx, dn, (1,),
                      mode=lax.GatherScatterMode.PROMISE_IN_BOUNDS)


def _make_sc_gat(H):
    mesh = plsc.VectorSubcoreMesh(core_axis_name="c", subcore_axis_name="s")

    @functools.partial(
        pl.kernel,
        out_type=(jax.ShapeDtypeStruct((2, N1, 128), _f32),
                  jax.ShapeDtypeStruct((2, N1, TW), _f32)),
        mesh=mesh,
        compiler_params=pltpu.CompilerParams(
            needs_layout_passes=False, use_tc_tiling_on_sc=False),
        scratch_types=[
            pltpu.VMEM((NCH, BLK), _i32),
            pltpu.VMEM((NCH, BLK), _i32),
            pltpu.VMEM((BLK, 128), _f32),
            pltpu.VMEM((BLK, 128), _f32),
            pltpu.VMEM((BLK, TW), _f32),
            pltpu.VMEM((BLK, TW), _f32),
            pltpu.VMEM((BLK, TW), _f32),
            pltpu.VMEM((BLK, TW), _f32),
            pltpu.VMEM((BLK, TW), _f32),
            pltpu.VMEM((BLK, TW), _f32),
            pltpu.VMEM_SHARED((N1, 128), _f32),
            pltpu.VMEM_SHARED((N1, TW), _f32),
            pltpu.SemaphoreType.DMA,
            pltpu.SemaphoreType.DMA,
        ],
    )
    def sc_gat(src_hbm, dst_hbm, asrc_hbm, adst_hbm, h_hbm,
               op_hbm, dp_hbm,
               sidx_v, didx_v, rows_a, rows_b, e_a, e_b, as_a, as_b,
               ad_a, ad_b, acc_sh, den_sh, sem_a, sem_b):
        core = lax.axis_index("c")
        sub = lax.axis_index("s")
        w = sub * 2 + core
        r0 = sub * RPT

        lane = lax.iota(_i32, 16)
        zero16 = jnp.zeros((16,), _f32)

        def zrow(r, c):
            for q in range(8):
                rows_a[r, pl.ds(q * 16, 16)] = zero16
            return c
        lax.fori_loop(0, BLK, zrow, 0)

        def zev(g, c):
            for h in range(TW):
                plsc.store_scatter(
                    e_a, [g * 16 + lane, jnp.full((16,), h, _i32)], zero16)
                plsc.store_scatter(
                    e_b, [g * 16 + lane, jnp.full((16,), h, _i32)], zero16)
            return c
        lax.fori_loop(0, BLK // 16, zev, 0)

        for t in range(RPT // BLK):
            pltpu.sync_copy(rows_a, acc_sh.at[pl.ds(r0 + t * BLK, BLK)])
            pltpu.sync_copy(e_a, den_sh.at[pl.ds(r0 + t * BLK, BLK)])
        rem = RPT % BLK
        if rem:
            t0 = r0 + (RPT // BLK) * BLK
            pltpu.sync_copy(rows_a.at[pl.ds(0, rem)], acc_sh.at[pl.ds(t0, rem)])
            pltpu.sync_copy(e_a.at[pl.ds(0, rem)], den_sh.at[pl.ds(t0, rem)])

        pltpu.sync_copy(src_hbm.at[w], sidx_v)
        pltpu.sync_copy(dst_hbm.at[w], didx_v)
        plsc.subcore_barrier()

        def fire(j, rows, asv, adv, sem):
            pltpu.async_copy(h_hbm.at[sidx_v.at[j]], rows, sem)
            pltpu.async_copy(asrc_hbm.at[sidx_v.at[j]], asv, sem)
            pltpu.async_copy(adst_hbm.at[didx_v.at[j]], adv, sem)

        def drain(j, rows, asv, adv, sem):
            pltpu.make_async_copy(h_hbm.at[sidx_v.at[j]], rows, sem).wait()
            pltpu.make_async_copy(asrc_hbm.at[sidx_v.at[j]], asv, sem).wait()
            pltpu.make_async_copy(adst_hbm.at[didx_v.at[j]], adv, sem).wait()

        def process(j, rows, ev_, asv, adv):
            def grp_body(g, c2):
                base = g * 16
                hsplat = [jnp.full((16,), h, _i32) for h in range(H)]
                evs = []
                for h in range(H):
                    av = plsc.load_gather(asv, [base + lane, hsplat[h]])
                    bv = plsc.load_gather(adv, [base + lane, hsplat[h]])
                    al = av + bv
                    al = jnp.where(al > 0, al, 0.2 * al)
                    ev = jnp.exp(al)
                    evs.append(ev)
                    plsc.store_scatter(ev_, [base + lane, hsplat[h]], ev)
                for i in range(16):
                    gi = base + i
                    for h in range(H):
                        sc = _bcast_lane(evs[h], i)
                        c0 = h * (128 // H)
                        for q in range(128 // H // 16):
                            cq = c0 + q * 16
                            rows[gi, pl.ds(cq, 16)] = (
                                rows[gi, pl.ds(cq, 16)] * sc)
                return c2

            lax.fori_loop(0, BLK // 16, grp_body, 0)
            pltpu.sync_copy(rows, acc_sh.at[didx_v.at[j]], add=True)
            pltpu.sync_copy(ev_, den_sh.at[didx_v.at[j]], add=True)

        fire(0, rows_a, as_a, ad_a, sem_a)

        def pair_body(jj, carry):
            j0 = jj * 2
            fire(j0 + 1, rows_b, as_b, ad_b, sem_b)
            drain(j0, rows_a, as_a, ad_a, sem_a)
            process(j0, rows_a, e_a, as_a, ad_a)

            @pl.when(j0 + 2 < NCH)
            def _():
                fire(j0 + 2, rows_a, as_a, ad_a, sem_a)

            drain(j0 + 1, rows_b, as_b, ad_b, sem_b)
            process(j0 + 1, rows_b, e_b, as_b, ad_b)
            return carry

        lax.fori_loop(0, NCH // 2, pair_body, 0)
        plsc.subcore_barrier()

        for t in range(RPT // BLK):
            rr = r0 + t * BLK
            pltpu.sync_copy(acc_sh.at[pl.ds(rr, BLK)], rows_a)
            pltpu.sync_copy(rows_a, op_hbm.at[core, pl.ds(rr, BLK)])
            pltpu.sync_copy(den_sh.at[pl.ds(rr, BLK)], e_a)
            pltpu.sync_copy(e_a, dp_hbm.at[core, pl.ds(rr, BLK)])
        if rem:
            t0 = r0 + (RPT // BLK) * BLK
            pltpu.sync_copy(acc_sh.at[pl.ds(t0, rem)], rows_a.at[pl.ds(0, rem)])
            pltpu.sync_copy(rows_a.at[pl.ds(0, rem)],
                            op_hbm.at[core, pl.ds(t0, rem)])
            pltpu.sync_copy(den_sh.at[pl.ds(t0, rem)], e_a.at[pl.ds(0, rem)])
            pltpu.sync_copy(e_a.at[pl.ds(0, rem)],
                            dp_hbm.at[core, pl.ds(t0, rem)])

    return sc_gat


_sc_gat4 = _make_sc_gat(4)
_sc_gat1 = _make_sc_gat(1)


_BR = 1264


def _lin0_body(x_ref, w_ref, ms_ref, md_ref, h_ref, as_ref, ad_ref):
    h = jnp.dot(x_ref[...], w_ref[...], preferred_element_type=_f32)
    h_ref[...] = h
    as_ref[...] = jnp.dot(h, ms_ref[...], preferred_element_type=_f32)
    ad_ref[...] = jnp.dot(h, md_ref[...], preferred_element_type=_f32)


def _mid_body(op_ref, dp_ref, b_ref, w_ref, ms_ref, md_ref, ex_ref,
              h_ref, as_ref, ad_ref):
    i = pl.program_id(0)
    num = op_ref[0] + op_ref[1]
    den = jnp.dot(dp_ref[0] + dp_ref[1], ex_ref[...],
                  preferred_element_type=_f32)
    s = num / (den + 1e-16) + b_ref[...]
    s = jnp.maximum(s, 0.0)
    rows = i * _BR + lax.broadcasted_iota(_i32, (_BR, 128), 0)
    s = jnp.where(rows < N, s, 0.0)
    h = jnp.dot(s, w_ref[...], preferred_element_type=_f32)
    h_ref[...] = h
    as_ref[...] = jnp.dot(h, ms_ref[...], preferred_element_type=_f32)
    ad_ref[...] = jnp.dot(h, md_ref[...], preferred_element_type=_f32)


def _fin_body(op_ref, dp_ref, b_ref, ex_ref, y_ref):
    num = op_ref[0] + op_ref[1]
    den = jnp.dot(dp_ref[0] + dp_ref[1], ex_ref[...],
                  preferred_element_type=_f32)
    y = num / (den + 1e-16) + b_ref[...]
    y_ref[...] = jnp.maximum(y, 0.0)


def _lin0(xpad, W0, Ms, Md):
    return pl.pallas_call(
        _lin0_body,
        grid=(N1 // _BR,),
        in_specs=[
            pl.BlockSpec((_BR, 128), lambda i: (i, 0)),
            pl.BlockSpec((128, 128), lambda i: (0, 0)),
            pl.BlockSpec((128, 8), lambda i: (0, 0)),
            pl.BlockSpec((128, 8), lambda i: (0, 0)),
        ],
        out_specs=[
            pl.BlockSpec((_BR, 128), lambda i: (i, 0)),
            pl.BlockSpec((_BR, 8), lambda i: (i, 0)),
            pl.BlockSpec((_BR, 8), lambda i: (i, 0)),
        ],
        out_shape=[
            jax.ShapeDtypeStruct((N1, 128), _f32),
            jax.ShapeDtypeStruct((N1, 8), _f32),
            jax.ShapeDtypeStruct((N1, 8), _f32),
        ],
    )(xpad, W0, Ms, Md)


def _mid(op, dp, b0, W1, Ms, Md, ex):
    return pl.pallas_call(
        _mid_body,
        grid=(N1 // _BR,),
        in_specs=[
            pl.BlockSpec((2, _BR, 128), lambda i: (0, i, 0)),
            pl.BlockSpec((2, _BR, 8), lambda i: (0, i, 0)),
            pl.BlockSpec((1, 128), lambda i: (0, 0)),
            pl.BlockSpec((128, 128), lambda i: (0, 0)),
            pl.BlockSpec((128, 8), lambda i: (0, 0)),
            pl.BlockSpec((128, 8), lambda i: (0, 0)),
            pl.BlockSpec((8, 128), lambda i: (0, 0)),
        ],
        out_specs=[
            pl.BlockSpec((_BR, 128), lambda i: (i, 0)),
            pl.BlockSpec((_BR, 8), lambda i: (i, 0)),
            pl.BlockSpec((_BR, 8), lambda i: (i, 0)),
        ],
        out_shape=[
            jax.ShapeDtypeStruct((N1, 128), _f32),
            jax.ShapeDtypeStruct((N1, 8), _f32),
            jax.ShapeDtypeStruct((N1, 8), _f32),
        ],
    )(op, dp, b0, W1, Ms, Md, ex)


def _fin(op, dp, b1, ex):
    return pl.pallas_call(
        _fin_body,
        grid=(10,),
        in_specs=[
            pl.BlockSpec((2, 1000, 128), lambda i: (0, i, 0)),
            pl.BlockSpec((2, 1000, 8), lambda i: (0, i, 0)),
            pl.BlockSpec((1, 128), lambda i: (0, 0)),
            pl.BlockSpec((8, 128), lambda i: (0, 0)),
        ],
        out_specs=pl.BlockSpec((1000, 128), lambda i: (i, 0)),
        out_shape=jax.ShapeDtypeStruct((N, 128), _f32),
    )(op, dp, b1, ex)


def kernel(x, edge_index, W0, a_src0, a_dst0, b0, W1, a_src1, a_dst1, b1):
    loops = jnp.arange(N, dtype=edge_index.dtype)
    pad = ETOT - (edge_index.shape[1] + N)
    src = jnp.concatenate(
        [edge_index[0], loops, jnp.zeros((pad,), edge_index.dtype)])
    dst = jnp.concatenate(
        [edge_index[1], loops, jnp.full((pad,), TRASH, edge_index.dtype)])
    src2d = src.reshape(NW, NCH, BLK).astype(_i32)
    dst2d = dst.reshape(NW, NCH, BLK).astype(_i32)

    xpad = jnp.concatenate([x, jnp.zeros((N1 - N, DIN), _f32)])

    hcol = jnp.repeat(jnp.arange(HEADS), HID)
    Ms0 = jnp.zeros((128, 8), _f32).at[jnp.arange(128), hcol].set(
        a_src0.reshape(-1))
    Md0 = jnp.zeros((128, 8), _f32).at[jnp.arange(128), hcol].set(
        a_dst0.reshape(-1))
    Ms1 = jnp.zeros((128, 8), _f32).at[:, 0].set(a_src1[0])
    Md1 = jnp.zeros((128, 8), _f32).at[:, 0].set(a_dst1[0])
    ex4 = jnp.zeros((8, 128), _f32).at[hcol, jnp.arange(128)].set(1.0)
    ex1 = jnp.zeros((8, 128), _f32).at[0].set(1.0)

    h0, as0, ad0 = _lin0(xpad, W0, Ms0, Md0)
    op0, dp0 = _sc_gat4(src2d, dst2d, as0, ad0, h0)

    h1, as1, ad1 = _mid(op0, dp0, b0.reshape(1, 128), W1, Ms1, Md1, ex4)
    op1, dp1 = _sc_gat1(src2d, dst2d, as1, ad1, h1)

    return _fin(op1, dp1, b1.reshape(1, 128), ex1)
